# baseline (speedup 1.0000x reference)
import functools

import jax
import jax.numpy as jnp
from jax import lax
from jax.experimental import pallas as pl
from jax.experimental.pallas import tpu as pltpu
from jax.experimental.pallas import tpu_sc as plsc

N = 10000
E = 320000
D = 128
H = 16
CP = 8

NC = 2
NS = 16

CH = 128
TOT = 2560
EP = TOT * CH
CHA = 48
CHB = 112
OFFB = NS * CHA
PAD_SRC = 0
PAD_DST = N

NT = 10240
ROWS_PER_SUB = NT // NS

RING = 4
LEAD = 2

_mesh = lambda: plsc.VectorSubcoreMesh(core_axis_name="c", subcore_axis_name="s")
_sc_params = lambda: pltpu.CompilerParams(use_tc_tiling_on_sc=False)


def _chunk_range(c, s):
    cb = jnp.where(c == 0, s * CHA, OFFB + s * CHB)
    nch = jnp.where(c == 0, CHA, CHB)
    return cb, nch, jnp.where(c == 0, CHA // RING, CHB // RING), \
        jnp.where(c == 0, CHA // 16, CHB // 16)


@functools.partial(
    pl.kernel,
    out_type=jax.ShapeDtypeStruct((NC, NS, ROWS_PER_SUB), jnp.float32),
    mesh=_mesh(),
    compiler_params=_sc_params(),
    scratch_types=[
        pltpu.VMEM((CHB, CH), jnp.int32),
        pltpu.VMEM((CH,), jnp.float32),
        pltpu.VMEM((ROWS_PER_SUB,), jnp.float32),
        pltpu.VMEM_SHARED((NT,), jnp.float32),
        pltpu.SemaphoreType.DMA,
    ],
)
def _sc_degree(dsti_hbm, out_hbm, idx_v, ones_v, buf_v, acc, sem):
    c = lax.axis_index("c")
    s = lax.axis_index("s")
    cb, _, _, nwave = _chunk_range(c, s)
    base = s * ROWS_PER_SUB
    pltpu.sync_copy(dsti_hbm.at[pl.ds(cb, CHB)], idx_v)
    for i in range(CH // 16):
        ones_v[pl.ds(i * 16, 16)] = jnp.full((16,), 1.0, jnp.float32)
    for i in range(ROWS_PER_SUB // 16):
        buf_v[pl.ds(i * 16, 16)] = jnp.full((16,), 1.0, jnp.float32)
    pltpu.sync_copy(buf_v, acc.at[pl.ds(base, ROWS_PER_SUB)])
    plsc.subcore_barrier()

    def wave(w, carry):
        for b in range(16):
            pltpu.async_copy(ones_v, acc.at[idx_v.at[w * 16 + b]], sem, add=True)
        for b in range(16):
            pltpu.make_async_copy(ones_v, acc.at[idx_v.at[w * 16 + b]], sem).wait()
        return carry

    lax.fori_loop(0, nwave, wave, 0)
    plsc.subcore_barrier()
    pltpu.sync_copy(acc.at[pl.ds(base, ROWS_PER_SUB)], buf_v)
    pltpu.sync_copy(buf_v, out_hbm.at[c, s])


def _make_sc_agg(F):
    @functools.partial(
        pl.kernel,
        out_type=jax.ShapeDtypeStruct((NC, NS, ROWS_PER_SUB, F), jnp.float32),
        mesh=_mesh(),
        compiler_params=_sc_params(),
        scratch_types=[
            pltpu.VMEM((CHB, CH), jnp.int32),
            pltpu.VMEM((CHB, CH), jnp.int32),
            [pltpu.VMEM((CH, F), jnp.float32)] * RING,
            pltpu.VMEM((ROWS_PER_SUB, F), jnp.float32),
            pltpu.VMEM_SHARED((NT, F), jnp.float32),
            [pltpu.SemaphoreType.DMA] * RING,
            [pltpu.SemaphoreType.DMA] * RING,
        ],
    )
    def agg(g_hbm, srci_hbm, dsti_hbm, out_hbm,
            src_v, dst_v, rows, buf_v, acc, gsem, ssem):
        c = lax.axis_index("c")
        s = lax.axis_index("s")
        cb, nch, ng, _ = _chunk_range(c, s)
        base = s * ROWS_PER_SUB
        pltpu.sync_copy(srci_hbm.at[pl.ds(cb, CHB)], src_v)
        pltpu.sync_copy(dsti_hbm.at[pl.ds(cb, CHB)], dst_v)
        pltpu.sync_copy(g_hbm.at[pl.ds(base, ROWS_PER_SUB)], buf_v)
        pltpu.sync_copy(buf_v, acc.at[pl.ds(base, ROWS_PER_SUB)])
        plsc.subcore_barrier()

        for b in range(LEAD):
            pltpu.async_copy(g_hbm.at[src_v.at[b]], rows[b], gsem[b])

        def group(i, carry):
            for b in range(RING):
                j = i * RING + b
                b2 = (b + LEAD) % RING
                pltpu.make_async_copy(g_hbm.at[src_v.at[j]], rows[b], gsem[b]).wait()
                pltpu.async_copy(rows[b], acc.at[dst_v.at[j]], ssem[b], add=True)

                @pl.when(j >= LEAD)
                def _():
                    pltpu.make_async_copy(
                        rows[b2], acc.at[dst_v.at[j - LEAD]], ssem[b2]).wait()

                @pl.when(j + LEAD < nch)
                def _():
                    pltpu.async_copy(g_hbm.at[src_v.at[j + LEAD]], rows[b2], gsem[b2])

            return carry

        lax.fori_loop(0, ng, group, 0)
        for k in range(LEAD):
            b = RING - LEAD + k
            pltpu.make_async_copy(rows[b], acc.at[dst_v.at[nch - LEAD + k]],
                                  ssem[b]).wait()
        plsc.subcore_barrier()
        pltpu.sync_copy(acc.at[pl.ds(base, ROWS_PER_SUB)], buf_v)
        pltpu.sync_copy(buf_v, out_hbm.at[c, s])

    return agg


_sc_agg16 = _make_sc_agg(H)
_sc_agg8 = _make_sc_agg(CP)

_BR = 2048
_GRID = NT // _BR


def _tc1_body(deg_ref, x_ref, w1_ref, g_ref, dinv_ref):
    d = deg_ref[0] + deg_ref[1] - 1.0
    dinv = lax.rsqrt(jnp.maximum(d, 1e-12))
    dinv_ref[...] = dinv
    g_ref[...] = jnp.dot(x_ref[...], w1_ref[...],
                         preferred_element_type=jnp.float32) * dinv[:, None]


def _tc1(deg2, x, w1):
    return pl.pallas_call(
        _tc1_body,
        grid=(_GRID,),
        in_specs=[
            pl.BlockSpec((2, _BR), lambda i: (0, i)),
            pl.BlockSpec((_BR, D), lambda i: (i, 0)),
            pl.BlockSpec((D, H), lambda i: (0, 0)),
        ],
        out_specs=[
            pl.BlockSpec((_BR, H), lambda i: (i, 0)),
            pl.BlockSpec((_BR,), lambda i: (i,)),
        ],
        out_shape=[
            jax.ShapeDtypeStruct((NT, H), jnp.float32),
            jax.ShapeDtypeStruct((NT,), jnp.float32),
        ],
    )(deg2, x, w1)


def _tc2_body(p_ref, g1_ref, dinv_ref, b1_ref, w2_ref, g2_ref):
    dinv = dinv_ref[...][:, None]
    pre = dinv * (p_ref[0] + p_ref[1] - g1_ref[...]) + b1_ref[...]
    h = jnp.maximum(pre, 0.0)
    g2_ref[...] = jnp.dot(h, w2_ref[...],
                          preferred_element_type=jnp.float32) * dinv


def _tc2(p, g1, dinv, b1, w2p):
    return pl.pallas_call(
        _tc2_body,
        grid=(_GRID,),
        in_specs=[
            pl.BlockSpec((2, _BR, H), lambda i: (0, i, 0)),
            pl.BlockSpec((_BR, H), lambda i: (i, 0)),
            pl.BlockSpec((_BR,), lambda i: (i,)),
            pl.BlockSpec((1, H), lambda i: (0, 0)),
            pl.BlockSpec((H, CP), lambda i: (0, 0)),
        ],
        out_specs=pl.BlockSpec((_BR, CP), lambda i: (i, 0)),
        out_shape=jax.ShapeDtypeStruct((NT, CP), jnp.float32),
    )(p, g1, dinv, b1, w2p)


def _tc3_body(p_ref, g2_ref, dinv_ref, b2_ref, out_ref):
    l = dinv_ref[...][:, None] * (p_ref[0] + p_ref[1] - g2_ref[...]) + b2_ref[...]
    col = lax.broadcasted_iota(jnp.int32, l.shape, 1)
    valid = col < 7
    m = jnp.max(jnp.where(valid, l, -jnp.inf), axis=1, keepdims=True)
    ssum = jnp.sum(jnp.where(valid, jnp.exp(l - m), 0.0), axis=1, keepdims=True)
    out_ref[...] = l - (jnp.log(ssum) + m)


def _tc3(p, g2, dinv, b2p):
    return pl.pallas_call(
        _tc3_body,
        grid=(_GRID,),
        in_specs=[
            pl.BlockSpec((2, _BR, CP), lambda i: (0, i, 0)),
            pl.BlockSpec((_BR, CP), lambda i: (i, 0)),
            pl.BlockSpec((_BR,), lambda i: (i,)),
            pl.BlockSpec((1, CP), lambda i: (0, 0)),
        ],
        out_specs=pl.BlockSpec((_BR, CP), lambda i: (i, 0)),
        out_shape=jax.ShapeDtypeStruct((N, CP), jnp.float32),
    )(p, g2, dinv, b2p)


@jax.jit
def kernel(x, edge_index, W1, b1, W2, b2):
    src = edge_index[0].astype(jnp.int32)
    dst = edge_index[1].astype(jnp.int32)
    npad = EP - E
    srcp = jnp.concatenate(
        [src, jnp.full((npad,), PAD_SRC, jnp.int32)]).reshape(TOT, CH)
    dstp = jnp.concatenate(
        [dst, jnp.full((npad,), PAD_DST, jnp.int32)]).reshape(TOT, CH)

    deg2 = _sc_degree(dstp).reshape(NC, NT)

    g1, dinv = _tc1(deg2, x, W1)
    p1 = _sc_agg16(g1, srcp, dstp).reshape(NC, NT, H)

    w2p = jnp.pad(W2, ((0, 0), (0, CP - 7)))
    g2 = _tc2(p1, g1, dinv, b1.reshape(1, H), w2p)
    p2 = _sc_agg8(g2, srcp, dstp).reshape(NC, NT, CP)

    b2p = jnp.pad(b2, (0, CP - 7)).reshape(1, CP)
    out = _tc3(p2, g2, dinv, b2p)
    return out[:, :7]


# speedup vs baseline: 1.1938x; 1.1938x over previous
import functools

import jax
import jax.numpy as jnp
from jax import lax
from jax.experimental import pallas as pl
from jax.experimental.pallas import tpu as pltpu
from jax.experimental.pallas import tpu_sc as plsc

N = 10000
E = 320000
D = 128
H = 16
CP = 8

NC = 2
NS = 16

CH = 128
TOT = 2560
CHMAX = 112
TOTP = TOT + CHMAX
EP = TOTP * CH
PAD_SRC = 0
PAD_DST = N

NT = 10240
ROWS_PER_SUB = NT // NS

RING = 4
LEAD = 2

_mesh = lambda: plsc.VectorSubcoreMesh(core_axis_name="c", subcore_axis_name="s")
_sc_params = lambda: pltpu.CompilerParams(use_tc_tiling_on_sc=False)


def _chunk_range(c, s, n0, n1):
    cb = jnp.where(c == 0, s * n0, NS * n0 + s * n1)
    nch = jnp.where(c == 0, n0, n1)
    return cb, nch, jnp.where(c == 0, n0 // RING, n1 // RING)


@functools.partial(
    pl.kernel,
    out_type=jax.ShapeDtypeStruct((NC, NS, ROWS_PER_SUB), jnp.float32),
    mesh=_mesh(),
    compiler_params=_sc_params(),
    scratch_types=[
        pltpu.VMEM((CHMAX, CH), jnp.int32),
        pltpu.VMEM((CH,), jnp.float32),
        pltpu.VMEM((ROWS_PER_SUB,), jnp.float32),
        pltpu.VMEM_SHARED((NT,), jnp.float32),
        pltpu.SemaphoreType.DMA,
    ],
)
def _sc_degree(dsti_hbm, out_hbm, idx_v, ones_v, buf_v, acc, sem):
    c = lax.axis_index("c")
    s = lax.axis_index("s")
    cb, _, _ = _chunk_range(c, s, 112, 48)
    nwave = jnp.where(c == 0, 112 // 16, 48 // 16)
    base = s * ROWS_PER_SUB
    pltpu.sync_copy(dsti_hbm.at[pl.ds(cb, CHMAX)], idx_v)
    for i in range(CH // 16):
        ones_v[pl.ds(i * 16, 16)] = jnp.full((16,), 1.0, jnp.float32)
    for i in range(ROWS_PER_SUB // 16):
        buf_v[pl.ds(i * 16, 16)] = jnp.full((16,), 1.0, jnp.float32)
    pltpu.sync_copy(buf_v, acc.at[pl.ds(base, ROWS_PER_SUB)])
    plsc.subcore_barrier()

    def wave(w, carry):
        for b in range(16):
            pltpu.async_copy(ones_v, acc.at[idx_v.at[w * 16 + b]], sem, add=True)
        for b in range(16):
            pltpu.make_async_copy(ones_v, acc.at[idx_v.at[w * 16 + b]], sem).wait()
        return carry

    lax.fori_loop(0, nwave, wave, 0)
    plsc.subcore_barrier()
    pltpu.sync_copy(acc.at[pl.ds(base, ROWS_PER_SUB)], buf_v)
    pltpu.sync_copy(buf_v, out_hbm.at[c, s])


def _make_sc_agg(F, n0, n1):
    @functools.partial(
        pl.kernel,
        out_type=jax.ShapeDtypeStruct((NC, NS, ROWS_PER_SUB, F), jnp.float32),
        mesh=_mesh(),
        compiler_params=_sc_params(),
        scratch_types=[
            pltpu.VMEM((CHMAX, CH), jnp.int32),
            pltpu.VMEM((CHMAX, CH), jnp.int32),
            [pltpu.VMEM((CH, F), jnp.float32)] * RING,
            pltpu.VMEM((ROWS_PER_SUB, F), jnp.float32),
            pltpu.VMEM_SHARED((NT, F), jnp.float32),
            [pltpu.SemaphoreType.DMA] * RING,
            [pltpu.SemaphoreType.DMA] * RING,
        ],
    )
    def agg(g_hbm, srci_hbm, dsti_hbm, out_hbm,
            src_v, dst_v, rows, buf_v, acc, gsem, ssem):
        c = lax.axis_index("c")
        s = lax.axis_index("s")
        cb, nch, ng = _chunk_range(c, s, n0, n1)
        base = s * ROWS_PER_SUB
        pltpu.sync_copy(srci_hbm.at[pl.ds(cb, CHMAX)], src_v)
        pltpu.sync_copy(dsti_hbm.at[pl.ds(cb, CHMAX)], dst_v)
        pltpu.sync_copy(g_hbm.at[pl.ds(base, ROWS_PER_SUB)], buf_v)
        pltpu.sync_copy(buf_v, acc.at[pl.ds(base, ROWS_PER_SUB)])
        plsc.subcore_barrier()

        for b in range(LEAD):
            pltpu.async_copy(g_hbm.at[src_v.at[b]], rows[b], gsem[b])

        def group(i, carry):
            for b in range(RING):
                j = i * RING + b
                b2 = (b + LEAD) % RING
                pltpu.make_async_copy(g_hbm.at[src_v.at[j]], rows[b], gsem[b]).wait()
                pltpu.async_copy(rows[b], acc.at[dst_v.at[j]], ssem[b], add=True)

                @pl.when(j >= LEAD)
                def _():
                    pltpu.make_async_copy(
                        rows[b2], acc.at[dst_v.at[j - LEAD]], ssem[b2]).wait()

                @pl.when(j + LEAD < nch)
                def _():
                    pltpu.async_copy(g_hbm.at[src_v.at[j + LEAD]], rows[b2], gsem[b2])

            return carry

        lax.fori_loop(0, ng, group, 0)
        for k in range(LEAD):
            b = RING - LEAD + k
            pltpu.make_async_copy(rows[b], acc.at[dst_v.at[nch - LEAD + k]],
                                  ssem[b]).wait()
        plsc.subcore_barrier()
        pltpu.sync_copy(acc.at[pl.ds(base, ROWS_PER_SUB)], buf_v)
        pltpu.sync_copy(buf_v, out_hbm.at[c, s])

    return agg


_sc_agg16 = _make_sc_agg(H, 108, 52)
_sc_agg8 = _make_sc_agg(CP, 92, 68)

_BR = 2048
_GRID = NT // _BR


def _tc1_body(deg_ref, x_ref, w1_ref, g_ref, dinv_ref):
    d = deg_ref[0] + deg_ref[1] - 1.0
    dinv = lax.rsqrt(jnp.maximum(d, 1e-12))
    dinv_ref[...] = dinv
    g_ref[...] = jnp.dot(x_ref[...], w1_ref[...],
                         preferred_element_type=jnp.float32) * dinv[:, None]


def _tc1(deg2, x, w1):
    return pl.pallas_call(
        _tc1_body,
        grid=(_GRID,),
        in_specs=[
            pl.BlockSpec((2, _BR), lambda i: (0, i)),
            pl.BlockSpec((_BR, D), lambda i: (i, 0)),
            pl.BlockSpec((D, H), lambda i: (0, 0)),
        ],
        out_specs=[
            pl.BlockSpec((_BR, H), lambda i: (i, 0)),
            pl.BlockSpec((_BR,), lambda i: (i,)),
        ],
        out_shape=[
            jax.ShapeDtypeStruct((NT, H), jnp.float32),
            jax.ShapeDtypeStruct((NT,), jnp.float32),
        ],
    )(deg2, x, w1)


def _tc2_body(p_ref, g1_ref, dinv_ref, b1_ref, w2_ref, g2_ref):
    dinv = dinv_ref[...][:, None]
    pre = dinv * (p_ref[0] + p_ref[1] - g1_ref[...]) + b1_ref[...]
    h = jnp.maximum(pre, 0.0)
    g2_ref[...] = jnp.dot(h, w2_ref[...],
                          preferred_element_type=jnp.float32) * dinv


def _tc2(p, g1, dinv, b1, w2p):
    return pl.pallas_call(
        _tc2_body,
        grid=(_GRID,),
        in_specs=[
            pl.BlockSpec((2, _BR, H), lambda i: (0, i, 0)),
            pl.BlockSpec((_BR, H), lambda i: (i, 0)),
            pl.BlockSpec((_BR,), lambda i: (i,)),
            pl.BlockSpec((1, H), lambda i: (0, 0)),
            pl.BlockSpec((H, CP), lambda i: (0, 0)),
        ],
        out_specs=pl.BlockSpec((_BR, CP), lambda i: (i, 0)),
        out_shape=jax.ShapeDtypeStruct((NT, CP), jnp.float32),
    )(p, g1, dinv, b1, w2p)


def _tc3_body(p_ref, g2_ref, dinv_ref, b2_ref, out_ref):
    l = dinv_ref[...][:, None] * (p_ref[0] + p_ref[1] - g2_ref[...]) + b2_ref[...]
    col = lax.broadcasted_iota(jnp.int32, l.shape, 1)
    valid = col < 7
    m = jnp.max(jnp.where(valid, l, -jnp.inf), axis=1, keepdims=True)
    ssum = jnp.sum(jnp.where(valid, jnp.exp(l - m), 0.0), axis=1, keepdims=True)
    out_ref[...] = l - (jnp.log(ssum) + m)


def _tc3(p, g2, dinv, b2p):
    return pl.pallas_call(
        _tc3_body,
        grid=(_GRID,),
        in_specs=[
            pl.BlockSpec((2, _BR, CP), lambda i: (0, i, 0)),
            pl.BlockSpec((_BR, CP), lambda i: (i, 0)),
            pl.BlockSpec((_BR,), lambda i: (i,)),
            pl.BlockSpec((1, CP), lambda i: (0, 0)),
        ],
        out_specs=pl.BlockSpec((_BR, CP), lambda i: (i, 0)),
        out_shape=jax.ShapeDtypeStruct((N, CP), jnp.float32),
    )(p, g2, dinv, b2p)


@jax.jit
def kernel(x, edge_index, W1, b1, W2, b2):
    src = edge_index[0].astype(jnp.int32)
    dst = edge_index[1].astype(jnp.int32)
    npad = EP - E
    srcp = jnp.concatenate(
        [src, jnp.full((npad,), PAD_SRC, jnp.int32)]).reshape(TOTP, CH)
    dstp = jnp.concatenate(
        [dst, jnp.full((npad,), PAD_DST, jnp.int32)]).reshape(TOTP, CH)

    deg2 = _sc_degree(dstp).reshape(NC, NT)

    g1, dinv = _tc1(deg2, x, W1)
    p1 = _sc_agg16(g1, srcp, dstp).reshape(NC, NT, H)

    w2p = jnp.pad(W2, ((0, 0), (0, CP - 7)))
    g2 = _tc2(p1, g1, dinv, b1.reshape(1, H), w2p)
    p2 = _sc_agg8(g2, srcp, dstp).reshape(NC, NT, CP)

    b2p = jnp.pad(b2, (0, CP - 7)).reshape(1, CP)
    out = _tc3(p2, g2, dinv, b2p)
    return out[:, :7]


# speedup vs baseline: 1.2219x; 1.0236x over previous
import functools

import jax
import jax.numpy as jnp
from jax import lax
from jax.experimental import pallas as pl
from jax.experimental.pallas import tpu as pltpu
from jax.experimental.pallas import tpu_sc as plsc

N = 10000
E = 320000
D = 128
H = 16
CP = 8

NC = 2
NS = 16

CH = 128
TOT = 2560
CHMAX = 120
TOTP = TOT + CHMAX
EP = TOTP * CH
PAD_SRC = 0
PAD_DST = N

NT = 10240
ROWS_PER_SUB = NT // NS

RING = 4
LEAD = 2

_mesh = lambda: plsc.VectorSubcoreMesh(core_axis_name="c", subcore_axis_name="s")
_sc_params = lambda: pltpu.CompilerParams(use_tc_tiling_on_sc=False)


def _chunk_range(c, s, n0, n1):
    cb = jnp.where(c == 0, s * n0, NS * n0 + s * n1)
    nch = jnp.where(c == 0, n0, n1)
    return cb, nch, jnp.where(c == 0, n0 // RING, n1 // RING)


@functools.partial(
    pl.kernel,
    out_type=jax.ShapeDtypeStruct((NC, NS, ROWS_PER_SUB), jnp.float32),
    mesh=_mesh(),
    compiler_params=_sc_params(),
    scratch_types=[
        pltpu.VMEM((CHMAX, CH), jnp.int32),
        pltpu.VMEM((CH,), jnp.float32),
        pltpu.VMEM((ROWS_PER_SUB,), jnp.float32),
        pltpu.VMEM_SHARED((NT,), jnp.float32),
        pltpu.SemaphoreType.DMA,
    ],
)
def _sc_degree(dsti_hbm, out_hbm, idx_v, ones_v, buf_v, acc, sem):
    c = lax.axis_index("c")
    s = lax.axis_index("s")
    nwave = jnp.where(c == 0, 112 // 16, 48 // 16)
    base = s * ROWS_PER_SUB

    @pl.when(c == 0)
    def _():
        pltpu.async_copy(dsti_hbm.at[pl.ds(s * 112, 112)],
                         idx_v.at[pl.ds(0, 112)], sem)

    @pl.when(c == 1)
    def _():
        pltpu.async_copy(dsti_hbm.at[pl.ds(NS * 112 + s * 48, 48)],
                         idx_v.at[pl.ds(0, 48)], sem)

    for i in range(CH // 16):
        ones_v[pl.ds(i * 16, 16)] = jnp.full((16,), 1.0, jnp.float32)
    for i in range(ROWS_PER_SUB // 16):
        buf_v[pl.ds(i * 16, 16)] = jnp.full((16,), 1.0, jnp.float32)

    @pl.when(c == 0)
    def _():
        pltpu.make_async_copy(dsti_hbm.at[pl.ds(s * 112, 112)],
                              idx_v.at[pl.ds(0, 112)], sem).wait()

    @pl.when(c == 1)
    def _():
        pltpu.make_async_copy(dsti_hbm.at[pl.ds(NS * 112 + s * 48, 48)],
                              idx_v.at[pl.ds(0, 48)], sem).wait()

    pltpu.sync_copy(buf_v, acc.at[pl.ds(base, ROWS_PER_SUB)])
    plsc.subcore_barrier()

    def wave(w, carry):
        for b in range(16):
            pltpu.async_copy(ones_v, acc.at[idx_v.at[w * 16 + b]], sem, add=True)
        for b in range(16):
            pltpu.make_async_copy(ones_v, acc.at[idx_v.at[w * 16 + b]], sem).wait()
        return carry

    lax.fori_loop(0, nwave, wave, 0)
    plsc.subcore_barrier()
    pltpu.sync_copy(acc.at[pl.ds(base, ROWS_PER_SUB)], buf_v)
    pltpu.sync_copy(buf_v, out_hbm.at[c, s])


def _make_sc_agg(F, n0, n1):
    @functools.partial(
        pl.kernel,
        out_type=jax.ShapeDtypeStruct((NC, NS, ROWS_PER_SUB, F), jnp.float32),
        mesh=_mesh(),
        compiler_params=_sc_params(),
        scratch_types=[
            pltpu.VMEM((CHMAX, CH), jnp.int32),
            pltpu.VMEM((CHMAX, CH), jnp.int32),
            [pltpu.VMEM((CH, F), jnp.float32)] * RING,
            pltpu.VMEM((ROWS_PER_SUB, F), jnp.float32),
            pltpu.VMEM_SHARED((NT, F), jnp.float32),
            [pltpu.SemaphoreType.DMA] * RING,
            [pltpu.SemaphoreType.DMA] * RING,
        ],
    )
    def agg(g_hbm, srci_hbm, dsti_hbm, out_hbm,
            src_v, dst_v, rows, buf_v, acc, gsem, ssem):
        c = lax.axis_index("c")
        s = lax.axis_index("s")
        cb, nch, ng = _chunk_range(c, s, n0, n1)
        base = s * ROWS_PER_SUB

        @pl.when(c == 0)
        def _():
            pltpu.async_copy(srci_hbm.at[pl.ds(s * n0, n0)],
                             src_v.at[pl.ds(0, n0)], gsem[0])
            pltpu.async_copy(dsti_hbm.at[pl.ds(s * n0, n0)],
                             dst_v.at[pl.ds(0, n0)], gsem[1])

        @pl.when(c == 1)
        def _():
            pltpu.async_copy(srci_hbm.at[pl.ds(NS * n0 + s * n1, n1)],
                             src_v.at[pl.ds(0, n1)], gsem[0])
            pltpu.async_copy(dsti_hbm.at[pl.ds(NS * n0 + s * n1, n1)],
                             dst_v.at[pl.ds(0, n1)], gsem[1])

        if F == 16:
            for r in range(ROWS_PER_SUB):
                buf_v[r, :] = jnp.zeros((16,), jnp.float32)
        else:
            pltpu.sync_copy(g_hbm.at[pl.ds(base, ROWS_PER_SUB)], buf_v)

        @pl.when(c == 0)
        def _():
            pltpu.make_async_copy(srci_hbm.at[pl.ds(s * n0, n0)],
                                  src_v.at[pl.ds(0, n0)], gsem[0]).wait()
            pltpu.make_async_copy(dsti_hbm.at[pl.ds(s * n0, n0)],
                                  dst_v.at[pl.ds(0, n0)], gsem[1]).wait()

        @pl.when(c == 1)
        def _():
            pltpu.make_async_copy(srci_hbm.at[pl.ds(NS * n0 + s * n1, n1)],
                                  src_v.at[pl.ds(0, n1)], gsem[0]).wait()
            pltpu.make_async_copy(dsti_hbm.at[pl.ds(NS * n0 + s * n1, n1)],
                                  dst_v.at[pl.ds(0, n1)], gsem[1]).wait()

        pltpu.sync_copy(buf_v, acc.at[pl.ds(base, ROWS_PER_SUB)])
        plsc.subcore_barrier()

        for b in range(LEAD):
            pltpu.async_copy(g_hbm.at[src_v.at[b]], rows[b], gsem[b])

        def group(i, carry):
            for b in range(RING):
                j = i * RING + b
                b2 = (b + LEAD) % RING
                pltpu.make_async_copy(g_hbm.at[src_v.at[j]], rows[b], gsem[b]).wait()
                pltpu.async_copy(rows[b], acc.at[dst_v.at[j]], ssem[b], add=True)

                @pl.when(j >= LEAD)
                def _():
                    pltpu.make_async_copy(
                        rows[b2], acc.at[dst_v.at[j - LEAD]], ssem[b2]).wait()

                @pl.when(j + LEAD < nch)
                def _():
                    pltpu.async_copy(g_hbm.at[src_v.at[j + LEAD]], rows[b2], gsem[b2])

            return carry

        lax.fori_loop(0, ng, group, 0)
        for k in range(LEAD):
            b = RING - LEAD + k
            pltpu.make_async_copy(rows[b], acc.at[dst_v.at[nch - LEAD + k]],
                                  ssem[b]).wait()
        plsc.subcore_barrier()
        pltpu.sync_copy(acc.at[pl.ds(base, ROWS_PER_SUB)], buf_v)
        pltpu.sync_copy(buf_v, out_hbm.at[c, s])

    return agg


_sc_agg16 = _make_sc_agg(H, 120, 40)
_sc_agg8 = _make_sc_agg(CP, 96, 64)

_BR = 2048
_GRID = NT // _BR


def _tc1_body(deg_ref, x_ref, w1_ref, g_ref, dinv_ref):
    d = deg_ref[0] + deg_ref[1] - 1.0
    dinv = lax.rsqrt(jnp.maximum(d, 1e-12))
    dinv_ref[...] = dinv
    g_ref[...] = jnp.dot(x_ref[...], w1_ref[...],
                         preferred_element_type=jnp.float32) * dinv[:, None]


def _tc1(deg2, x, w1):
    return pl.pallas_call(
        _tc1_body,
        grid=(_GRID,),
        in_specs=[
            pl.BlockSpec((2, _BR), lambda i: (0, i)),
            pl.BlockSpec((_BR, D), lambda i: (i, 0)),
            pl.BlockSpec((D, H), lambda i: (0, 0)),
        ],
        out_specs=[
            pl.BlockSpec((_BR, H), lambda i: (i, 0)),
            pl.BlockSpec((_BR,), lambda i: (i,)),
        ],
        out_shape=[
            jax.ShapeDtypeStruct((NT, H), jnp.float32),
            jax.ShapeDtypeStruct((NT,), jnp.float32),
        ],
    )(deg2, x, w1)


def _tc2_body(p_ref, g1_ref, dinv_ref, b1_ref, w2_ref, g2_ref):
    dinv = dinv_ref[...][:, None]
    pre = dinv * (p_ref[0] + p_ref[1] + g1_ref[...]) + b1_ref[...]
    h = jnp.maximum(pre, 0.0)
    g2_ref[...] = jnp.dot(h, w2_ref[...],
                          preferred_element_type=jnp.float32) * dinv


def _tc2(p, g1, dinv, b1, w2p):
    return pl.pallas_call(
        _tc2_body,
        grid=(_GRID,),
        in_specs=[
            pl.BlockSpec((2, _BR, H), lambda i: (0, i, 0)),
            pl.BlockSpec((_BR, H), lambda i: (i, 0)),
            pl.BlockSpec((_BR,), lambda i: (i,)),
            pl.BlockSpec((1, H), lambda i: (0, 0)),
            pl.BlockSpec((H, CP), lambda i: (0, 0)),
        ],
        out_specs=pl.BlockSpec((_BR, CP), lambda i: (i, 0)),
        out_shape=jax.ShapeDtypeStruct((NT, CP), jnp.float32),
    )(p, g1, dinv, b1, w2p)


def _tc3_body(p_ref, g2_ref, dinv_ref, b2_ref, out_ref):
    l = dinv_ref[...][:, None] * (p_ref[0] + p_ref[1] - g2_ref[...]) + b2_ref[...]
    col = lax.broadcasted_iota(jnp.int32, l.shape, 1)
    valid = col < 7
    m = jnp.max(jnp.where(valid, l, -jnp.inf), axis=1, keepdims=True)
    ssum = jnp.sum(jnp.where(valid, jnp.exp(l - m), 0.0), axis=1, keepdims=True)
    out_ref[...] = l - (jnp.log(ssum) + m)


def _tc3(p, g2, dinv, b2p):
    return pl.pallas_call(
        _tc3_body,
        grid=(_GRID,),
        in_specs=[
            pl.BlockSpec((2, _BR, CP), lambda i: (0, i, 0)),
            pl.BlockSpec((_BR, CP), lambda i: (i, 0)),
            pl.BlockSpec((_BR,), lambda i: (i,)),
            pl.BlockSpec((1, CP), lambda i: (0, 0)),
        ],
        out_specs=pl.BlockSpec((_BR, CP), lambda i: (i, 0)),
        out_shape=jax.ShapeDtypeStruct((N, CP), jnp.float32),
    )(p, g2, dinv, b2p)


@jax.jit
def kernel(x, edge_index, W1, b1, W2, b2):
    src = edge_index[0].astype(jnp.int32)
    dst = edge_index[1].astype(jnp.int32)
    npad = EP - E
    srcp = jnp.concatenate(
        [src, jnp.full((npad,), PAD_SRC, jnp.int32)]).reshape(TOTP, CH)
    dstp = jnp.concatenate(
        [dst, jnp.full((npad,), PAD_DST, jnp.int32)]).reshape(TOTP, CH)

    deg2 = _sc_degree(dstp).reshape(NC, NT)

    g1, dinv = _tc1(deg2, x, W1)
    p1 = _sc_agg16(g1, srcp, dstp).reshape(NC, NT, H)

    w2p = jnp.pad(W2, ((0, 0), (0, CP - 7)))
    g2 = _tc2(p1, g1, dinv, b1.reshape(1, H), w2p)
    p2 = _sc_agg8(g2, srcp, dstp).reshape(NC, NT, CP)

    b2p = jnp.pad(b2, (0, CP - 7)).reshape(1, CP)
    out = _tc3(p2, g2, dinv, b2p)
    return out[:, :7]
